# baseline (speedup 1.0000x reference)
import dataclasses
import functools

import jax
import jax.numpy as jnp
from jax import lax
from jax.experimental import pallas as pl
from jax.experimental.pallas import tpu as pltpu
from jax.experimental.pallas import tpu_sc as plsc

_D = 64
_S = 200
_NC = 2
_NS = 16
_NW = _NC * _NS
_BB = 8
_BPB = 128
_SR = _NW // _BB
_SPT = _S // _SR
_NB = 2


def _compiler_params():
    cp = pltpu.CompilerParams(use_tc_tiling_on_sc=False)
    if "needs_layout_passes" in pltpu.CompilerParams.__dataclass_fields__:
        cp = dataclasses.replace(cp, needs_layout_passes=False)
    return cp


def _tpe_sc(x, emb, pos):
    mesh = plsc.VectorSubcoreMesh(core_axis_name="c", subcore_axis_name="s")

    @functools.partial(
        pl.kernel,
        mesh=mesh,
        compiler_params=_compiler_params(),
        out_type=jax.ShapeDtypeStruct((_S, _D // 8, _BB, 8, _BPB),
                                      jnp.float32),
        scratch_types=[
            pltpu.VMEM((_SPT, _D), jnp.float32),
            pltpu.VMEM((_BPB, _S), jnp.int32),
            pltpu.VMEM((_SPT, _BPB), jnp.int32),
            pltpu.VMEM((_NB, _BPB, _D), jnp.float32),
            pltpu.VMEM((_NB, _D, _BPB), jnp.float32),
            pltpu.VMEM((_BPB, 16), jnp.int32),
            pltpu.SemaphoreType.DMA((_NB,)),
            pltpu.SemaphoreType.DMA((_NB,)),
        ],
    )
    def k(emb_hbm, x_hbm, pos_hbm, out5, pos_v, xblk, idx_t, rows, stg,
          cidx, gsem, osem):
        wid = lax.axis_index("s") * _NC + lax.axis_index("c")
        bb = wid // _SR
        sr = wid % _SR
        iota = lax.iota(jnp.int32, 16)
        drows = [dseg * 16 + iota for dseg in range(_D // 16)]

        pltpu.sync_copy(x_hbm.at[pl.ds(bb * _BPB, _BPB)], xblk)
        pltpu.sync_copy(pos_hbm.at[pl.ds(sr * _SPT, _SPT)], pos_v)

        @pl.loop(0, _BPB)
        def _(b):
            cidx.at[b][...] = jnp.full((16,), b, jnp.int32)

        @pl.loop(0, _SPT)
        def _(s):
            col = jnp.full((16,), sr * _SPT + s, jnp.int32)
            for bseg in range(_BPB // 16):
                v = plsc.load_gather(xblk, [bseg * 16 + iota, col])
                idx_t.at[s].at[pl.ds(bseg * 16, 16)][...] = v

        def start_gather(s, j):
            pltpu.async_copy(emb_hbm.at[idx_t.at[s]], rows.at[j], gsem.at[j])

        for j in range(_NB):
            start_gather(j, j)

        @pl.loop(0, _SPT, step=_NB)
        def _(c):
            for j in range(_NB):
                s = c + j
                pltpu.make_async_copy(emb_hbm.at[pl.ds(0, _BPB)], rows.at[j],
                                      gsem.at[j]).wait()

                @pl.when(s >= _NB)
                def _():
                    for _tr in range(_D // 8):
                        pltpu.make_async_copy(stg.at[j].at[pl.ds(0, 8)],
                                              out5.at[0, 0, 0],
                                              osem.at[j]).wait()

                for dseg in range(_D // 16):
                    pos_seg = pos_v.at[s].at[pl.ds(dseg * 16, 16)][...]
                    dr = drows[dseg]

                    @plsc.parallel_loop(0, _BPB, unroll=4)
                    def _(b):
                        v = rows.at[j].at[b].at[pl.ds(dseg * 16, 16)][...]
                        bcol = cidx.at[b][...]
                        plsc.store_scatter(stg.at[j], [dr, bcol],
                                           v + pos_seg)

                s_glob = sr * _SPT + s
                for tr in range(_D // 8):
                    pltpu.async_copy(stg.at[j].at[pl.ds(tr * 8, 8)],
                                     out5.at[s_glob, tr, bb], osem.at[j])

                @pl.when(s + _NB < _SPT)
                def _():
                    start_gather(s + _NB, j)

        for j in range(_NB):
            for _tr in range(_D // 8):
                pltpu.make_async_copy(stg.at[j].at[pl.ds(0, 8)],
                                      out5.at[0, 0, 0], osem.at[j]).wait()

    return k(emb, x, pos)


def kernel(x, embedding, pos_embedding):
    out5 = _tpe_sc(x.astype(jnp.int32), embedding, pos_embedding)
    return out5.transpose(2, 4, 0, 1, 3).reshape(_BB * _BPB, _S, _D)


# speedup vs baseline: 1.7005x; 1.7005x over previous
import dataclasses
import functools

import jax
import jax.numpy as jnp
from jax import lax
from jax.experimental import pallas as pl
from jax.experimental.pallas import tpu as pltpu
from jax.experimental.pallas import tpu_sc as plsc

_D = 64
_S = 200
_B = 1024
_NC = 2
_NS = 16
_NW = _NC * _NS
_G0 = 128
_G1 = _S - _G0
_NB = 2
_UP = 104


def _compiler_params():
    cp = pltpu.CompilerParams(use_tc_tiling_on_sc=False)
    if "needs_layout_passes" in pltpu.CompilerParams.__dataclass_fields__:
        cp = dataclasses.replace(cp, needs_layout_passes=False)
    return cp


def _sc_gather(x, emb):
    spt = _B // _NW
    mesh = plsc.VectorSubcoreMesh(core_axis_name="c", subcore_axis_name="s")

    @functools.partial(
        pl.kernel,
        mesh=mesh,
        compiler_params=_compiler_params(),
        out_type=jax.ShapeDtypeStruct((_B * 2 * _UP, _D), jnp.float32),
        scratch_types=[
            pltpu.VMEM((spt, _S), jnp.int32),
            pltpu.VMEM((_NB, _S, _D), jnp.float32),
            pltpu.SemaphoreType.DMA((_NB,)),
            pltpu.SemaphoreType.DMA((_NB,)),
        ],
    )
    def k(emb_hbm, x_hbm, y_hbm, idx_all, rows, gsem, osem):
        wid = lax.axis_index("s") * _NC + lax.axis_index("c")
        seq0 = wid * spt
        pltpu.sync_copy(x_hbm.at[pl.ds(seq0, spt)], idx_all)

        def start_gather(nloc, j):
            pltpu.async_copy(emb_hbm.at[idx_all.at[nloc, pl.ds(0, _G0)]],
                             rows.at[j].at[pl.ds(0, _G0)], gsem.at[j])
            pltpu.async_copy(emb_hbm.at[idx_all.at[nloc, pl.ds(_G0, _G1)]],
                             rows.at[j].at[pl.ds(_G0, _G1)], gsem.at[j])

        for j in range(_NB):
            start_gather(j, j)

        @pl.loop(0, spt, step=_NB)
        def _(c):
            for j in range(_NB):
                nloc = c + j
                pltpu.make_async_copy(emb_hbm.at[pl.ds(0, _S)], rows.at[j],
                                      gsem.at[j]).wait()
                pltpu.async_copy(rows.at[j],
                                 y_hbm.at[pl.ds((seq0 + nloc) * 2 * _UP, _S)],
                                 osem.at[j])

                @pl.when(nloc + _NB < spt)
                def _():
                    pltpu.make_async_copy(rows.at[j], y_hbm.at[pl.ds(0, _S)],
                                          osem.at[j]).wait()
                    start_gather(nloc + _NB, j)

        for j in range(_NB):
            pltpu.make_async_copy(rows.at[j], y_hbm.at[pl.ds(0, _S)],
                                  osem.at[j]).wait()

    return k(emb, x)


def _tc_relayout(y3, posr):
    def body(y_ref, p_ref, o_ref):
        for u in range(_S // 2):
            xb = y_ref[:, u, :] + p_ref[u, :]
            o_ref[pl.ds(2 * u, 2)] = xb.T.reshape(2, 8, 1, 8, 128)

    return pl.pallas_call(
        body,
        grid=(_B // 128,),
        in_specs=[
            pl.BlockSpec((128, _UP, 128), lambda tc: (tc, 0, 0)),
            pl.BlockSpec((_S // 2, 128), lambda tc: (0, 0)),
        ],
        out_specs=pl.BlockSpec((_S, _D // 8, 1, 8, 128),
                               lambda tc: (0, 0, tc, 0, 0)),
        out_shape=jax.ShapeDtypeStruct((_S, _D // 8, _B // 128, 8, 128),
                                       jnp.float32),
    )(y3, posr)


def kernel(x, embedding, pos_embedding):
    y = _sc_gather(x.astype(jnp.int32), embedding)
    y3 = y.reshape(_B, _UP, 2 * _D)
    posr = pos_embedding.reshape(_S // 2, 2 * _D)
    out5 = _tc_relayout(y3, posr)
    return out5.transpose(2, 4, 0, 1, 3).reshape(_B, _S, _D)


# speedup vs baseline: 1.7013x; 1.0005x over previous
import dataclasses
import functools

import jax
import jax.numpy as jnp
from jax import lax
from jax.experimental import pallas as pl
from jax.experimental.pallas import tpu as pltpu
from jax.experimental.pallas import tpu_sc as plsc

_D = 64
_S = 200
_B = 1024
_NC = 2
_NS = 16
_NW = _NC * _NS
_G0 = 128
_G1 = _S - _G0
_NB = 2
_UP = 104


def _compiler_params():
    cp = pltpu.CompilerParams(use_tc_tiling_on_sc=False,
                              skip_device_barrier=True)
    if "needs_layout_passes" in pltpu.CompilerParams.__dataclass_fields__:
        cp = dataclasses.replace(cp, needs_layout_passes=False)
    return cp


def _sc_gather(x, emb):
    spt = _B // _NW
    mesh = plsc.VectorSubcoreMesh(core_axis_name="c", subcore_axis_name="s")

    @functools.partial(
        pl.kernel,
        mesh=mesh,
        compiler_params=_compiler_params(),
        out_type=jax.ShapeDtypeStruct((_B * 2 * _UP, _D), jnp.float32),
        scratch_types=[
            pltpu.VMEM((spt, _S), jnp.int32),
            pltpu.VMEM((_NB, _S, _D), jnp.float32),
            pltpu.SemaphoreType.DMA((_NB,)),
            pltpu.SemaphoreType.DMA((_NB,)),
        ],
    )
    def k(emb_hbm, x_hbm, y_hbm, idx_all, rows, gsem, osem):
        wid = lax.axis_index("s") * _NC + lax.axis_index("c")
        seq0 = wid * spt
        pltpu.sync_copy(x_hbm.at[pl.ds(seq0, spt)], idx_all)

        def start_gather(nloc, j):
            pltpu.async_copy(emb_hbm.at[idx_all.at[nloc, pl.ds(0, _G0)]],
                             rows.at[j].at[pl.ds(0, _G0)], gsem.at[j])
            pltpu.async_copy(emb_hbm.at[idx_all.at[nloc, pl.ds(_G0, _G1)]],
                             rows.at[j].at[pl.ds(_G0, _G1)], gsem.at[j])

        for j in range(_NB):
            start_gather(j, j)

        @pl.loop(0, spt, step=_NB)
        def _(c):
            for j in range(_NB):
                nloc = c + j
                pltpu.make_async_copy(emb_hbm.at[pl.ds(0, _S)], rows.at[j],
                                      gsem.at[j]).wait()
                pltpu.async_copy(rows.at[j],
                                 y_hbm.at[pl.ds((seq0 + nloc) * 2 * _UP, _S)],
                                 osem.at[j])

                @pl.when(nloc + _NB < spt)
                def _():
                    pltpu.make_async_copy(rows.at[j], y_hbm.at[pl.ds(0, _S)],
                                          osem.at[j]).wait()
                    start_gather(nloc + _NB, j)

        for j in range(_NB):
            pltpu.make_async_copy(rows.at[j], y_hbm.at[pl.ds(0, _S)],
                                  osem.at[j]).wait()

    return k(emb, x)


def _tc_relayout(y3, posr):
    def body(y_ref, p_ref, o_ref):
        for u in range(_S // 2):
            xb = y_ref[:, u, :] + p_ref[u, :]
            o_ref[pl.ds(2 * u, 2)] = xb.T.reshape(2, 8, 1, 8, 128)

    return pl.pallas_call(
        body,
        grid=(_B // 128,),
        in_specs=[
            pl.BlockSpec((128, _UP, 128), lambda tc: (tc, 0, 0)),
            pl.BlockSpec((_S // 2, 128), lambda tc: (0, 0)),
        ],
        out_specs=pl.BlockSpec((_S, _D // 8, 1, 8, 128),
                               lambda tc: (0, 0, tc, 0, 0)),
        out_shape=jax.ShapeDtypeStruct((_S, _D // 8, _B // 128, 8, 128),
                                       jnp.float32),
    )(y3, posr)


def kernel(x, embedding, pos_embedding):
    y = _sc_gather(x.astype(jnp.int32), embedding)
    y3 = y.reshape(_B, _UP, 2 * _D)
    posr = pos_embedding.reshape(_S // 2, 2 * _D)
    out5 = _tc_relayout(y3, posr)
    return out5.transpose(2, 4, 0, 1, 3).reshape(_B, _S, _D)


# speedup vs baseline: 1.7152x; 1.0082x over previous
import dataclasses
import functools

import jax
import jax.numpy as jnp
from jax import lax
from jax.experimental import pallas as pl
from jax.experimental.pallas import tpu as pltpu
from jax.experimental.pallas import tpu_sc as plsc

_D = 64
_S = 200
_B = 1024
_NC = 2
_NS = 16
_NW = _NC * _NS
_G0 = 128
_G1 = _S - _G0
_NB = 4
_UP = 104


def _compiler_params():
    cp = pltpu.CompilerParams(use_tc_tiling_on_sc=False)
    if "needs_layout_passes" in pltpu.CompilerParams.__dataclass_fields__:
        cp = dataclasses.replace(cp, needs_layout_passes=False)
    return cp


def _sc_gather(x, emb):
    spt = _B // _NW
    mesh = plsc.VectorSubcoreMesh(core_axis_name="c", subcore_axis_name="s")

    @functools.partial(
        pl.kernel,
        mesh=mesh,
        compiler_params=_compiler_params(),
        out_type=jax.ShapeDtypeStruct((_B * 2 * _UP, _D), jnp.float32),
        scratch_types=[
            pltpu.VMEM((spt, _S), jnp.int32),
            pltpu.VMEM((_NB, _S, _D), jnp.float32),
            pltpu.SemaphoreType.DMA((_NB,)),
            pltpu.SemaphoreType.DMA((_NB,)),
        ],
    )
    def k(emb_hbm, x_hbm, y_hbm, idx_all, rows, gsem, osem):
        wid = lax.axis_index("s") * _NC + lax.axis_index("c")
        seq0 = wid * spt
        pltpu.sync_copy(x_hbm.at[pl.ds(seq0, spt)], idx_all)

        def start_gather(nloc, j):
            pltpu.async_copy(emb_hbm.at[idx_all.at[nloc, pl.ds(0, _G0)]],
                             rows.at[j].at[pl.ds(0, _G0)], gsem.at[j])
            pltpu.async_copy(emb_hbm.at[idx_all.at[nloc, pl.ds(_G0, _G1)]],
                             rows.at[j].at[pl.ds(_G0, _G1)], gsem.at[j])

        for j in range(_NB):
            start_gather(j, j)

        @pl.loop(0, spt, step=_NB)
        def _(c):
            for j in range(_NB):
                nloc = c + j
                pltpu.make_async_copy(emb_hbm.at[pl.ds(0, _S)], rows.at[j],
                                      gsem.at[j]).wait()
                pltpu.async_copy(rows.at[j],
                                 y_hbm.at[pl.ds((seq0 + nloc) * 2 * _UP, _S)],
                                 osem.at[j])

                @pl.when(nloc + _NB < spt)
                def _():
                    pltpu.make_async_copy(rows.at[j], y_hbm.at[pl.ds(0, _S)],
                                          osem.at[j]).wait()
                    start_gather(nloc + _NB, j)

        for j in range(_NB):
            pltpu.make_async_copy(rows.at[j], y_hbm.at[pl.ds(0, _S)],
                                  osem.at[j]).wait()

    return k(emb, x)


def _tc_relayout(y3, posr):
    def body(y_ref, p_ref, o_ref):
        for u in range(_S // 2):
            xb = y_ref[:, u, :] + p_ref[u, :]
            o_ref[pl.ds(2 * u, 2)] = xb.T.reshape(2, 8, 1, 8, 128)

    return pl.pallas_call(
        body,
        grid=(_B // 128,),
        in_specs=[
            pl.BlockSpec((128, _UP, 128), lambda tc: (tc, 0, 0)),
            pl.BlockSpec((_S // 2, 128), lambda tc: (0, 0)),
        ],
        out_specs=pl.BlockSpec((_S, _D // 8, 1, 8, 128),
                               lambda tc: (0, 0, tc, 0, 0)),
        out_shape=jax.ShapeDtypeStruct((_S, _D // 8, _B // 128, 8, 128),
                                       jnp.float32),
    )(y3, posr)


def kernel(x, embedding, pos_embedding):
    y = _sc_gather(x.astype(jnp.int32), embedding)
    y3 = y.reshape(_B, _UP, 2 * _D)
    posr = pos_embedding.reshape(_S // 2, 2 * _D)
    out5 = _tc_relayout(y3, posr)
    return out5.transpose(2, 4, 0, 1, 3).reshape(_B, _S, _D)
